# Initial kernel scaffold; baseline (speedup 1.0000x reference)
import functools

import jax
import jax.numpy as jnp
from jax import lax
from jax.experimental import pallas as pl
from jax.experimental.pallas import tpu as pltpu
from jax.experimental.pallas import tpu_sc as plsc

B = 16384
SL = 200
D = 32
PROJ = 128
NC = 2
NS = 16
NW = NC * NS
SEQ_PER_W = B // NW
C = 16
NCHUNK = SEQ_PER_W // C


def _sc_pool(ids_flat, emb):
    mesh = plsc.VectorSubcoreMesh(core_axis_name="c", subcore_axis_name="s")

    @functools.partial(
        pl.kernel,
        out_type=(
            jax.ShapeDtypeStruct((B, D), jnp.float32),
            jax.ShapeDtypeStruct((B, 16), jnp.float32),
        ),
        mesh=mesh,
        scratch_types=[
            pltpu.VMEM((C * SL + 16,), jnp.int32),
            pltpu.VMEM((SL, D), jnp.float32),
            pltpu.VMEM((C, D), jnp.float32),
            pltpu.VMEM((C, 16), jnp.float32),
            pltpu.SemaphoreType.DMA,
        ],
    )
    def k(ids_hbm, emb_hbm, sums_hbm, cnts_hbm, ids_v, rows_v, sums_b, cnts_b, sem):
        wid = lax.axis_index("c") * NS + lax.axis_index("s")
        lane = lax.iota(jnp.int32, 16)
        zeros = jnp.zeros((16,), jnp.float32)

        def chunk_body(i, _):
            base_seq = wid * SEQ_PER_W + i * C
            pltpu.sync_copy(
                ids_hbm.at[pl.ds(base_seq * SL, C * SL)],
                ids_v.at[pl.ds(0, C * SL)],
            )

            def seq_body(s, _):
                off = s * SL
                cp0 = pltpu.async_copy(
                    emb_hbm.at[ids_v.at[pl.ds(off, 128)]],
                    rows_v.at[pl.ds(0, 128)], sem)
                cp1 = pltpu.async_copy(
                    emb_hbm.at[ids_v.at[pl.ds(off + 128, SL - 128)]],
                    rows_v.at[pl.ds(128, SL - 128)], sem)
                cp0.wait()
                cp1.wait()

                def red_body(j, carry):
                    a0, a1 = carry
                    r = j * 4
                    for u in range(4):
                        a0 = a0 + rows_v[r + u, pl.ds(0, 16)]
                        a1 = a1 + rows_v[r + u, pl.ds(16, 16)]
                    return a0, a1

                a0, a1 = lax.fori_loop(0, SL // 4, red_body, (zeros, zeros))

                def cnt_body(kk, cv):
                    ids16 = ids_v[pl.ds(off + kk * 16, 16)]
                    return cv + jnp.where(ids16 != 0, 1.0, 0.0).astype(jnp.float32)

                cv = lax.fori_loop(0, SL // 16, cnt_body, zeros)
                ids_tail = ids_v[pl.ds(off + (SL // 16) * 16, 16)]
                tail_ok = (lane < (SL - (SL // 16) * 16)) & (ids_tail != 0)
                cv = cv + jnp.where(tail_ok, 1.0, 0.0).astype(jnp.float32)
                cnt = jnp.sum(cv)

                sums_b[s, pl.ds(0, 16)] = a0
                sums_b[s, pl.ds(16, 16)] = a1
                cnts_b[s, pl.ds(0, 16)] = jnp.full((16,), cnt, jnp.float32)
                return 0

            lax.fori_loop(0, C, seq_body, 0)
            pltpu.sync_copy(sums_b, sums_hbm.at[pl.ds(base_seq, C)])
            pltpu.sync_copy(cnts_b, cnts_hbm.at[pl.ds(base_seq, C)])
            return 0

        lax.fori_loop(0, NCHUNK, chunk_body, 0)

    return k(ids_flat, emb)


def _tc_project(sums, cnts, W, b2d):
    BLK = 2048

    def body(s_ref, c_ref, w_ref, b_ref, o_ref):
        s = s_ref[...]
        cnt = jnp.max(c_ref[...], axis=1, keepdims=True)
        acc = jnp.dot(s, w_ref[...], preferred_element_type=jnp.float32)
        o_ref[...] = acc / (cnt + 1e-10) + b_ref[...]

    return pl.pallas_call(
        body,
        grid=(B // BLK,),
        in_specs=[
            pl.BlockSpec((BLK, D), lambda i: (i, 0)),
            pl.BlockSpec((BLK, 16), lambda i: (i, 0)),
            pl.BlockSpec((D, PROJ), lambda i: (0, 0)),
            pl.BlockSpec((1, PROJ), lambda i: (0, 0)),
        ],
        out_specs=pl.BlockSpec((BLK, PROJ), lambda i: (i, 0)),
        out_shape=jax.ShapeDtypeStruct((B, PROJ), jnp.float32),
    )(sums, cnts, W, b2d)


def kernel(input_ids, emb, W, b):
    ids_flat = input_ids.astype(jnp.int32).reshape(-1)
    sums, cnts = _sc_pool(ids_flat, emb)
    return _tc_project(sums, cnts, W, b.reshape(1, PROJ))


# speedup vs baseline: 39.3370x; 39.3370x over previous
import functools

import jax
import jax.numpy as jnp
from jax import lax
from jax.experimental import pallas as pl
from jax.experimental.pallas import tpu as pltpu
from jax.experimental.pallas import tpu_sc as plsc

B = 16384
SL = 200
D = 32
PROJ = 128
V = 1000000
NC = 2
NS = 16
NW = NC * NS
SEQ_PER_W = B // NW
C = 128
NCHUNK = SEQ_PER_W // C
OUTW = 128
RD = 8
VBK = 65536
OBK = VBK // 4


def _tc_relayout(emb_t):
    grid = -(-V // VBK)

    def body(x_ref, o_ref):
        r = lax.broadcasted_iota(jnp.int32, (PROJ, PROJ), 0)
        c = lax.broadcasted_iota(jnp.int32, (PROJ, PROJ), 1)
        eye = jnp.where(r == c, 1.0, 0.0).astype(jnp.float32)
        xx = jnp.concatenate(
            [x_ref[:, pl.ds(q * OBK, OBK)] for q in range(4)], axis=0)
        o_ref[...] = lax.dot_general(
            xx, eye, (((0,), (0,)), ((), ())),
            preferred_element_type=jnp.float32)

    return pl.pallas_call(
        body,
        grid=(grid,),
        in_specs=[pl.BlockSpec((D, VBK), lambda i: (0, i))],
        out_specs=pl.BlockSpec((OBK, PROJ), lambda i: (i, 0)),
        out_shape=jax.ShapeDtypeStruct((-(-V // VBK) * OBK, PROJ), jnp.float32),
    )(emb_t)


def _sc_pool(ids4, emb_lin):
    mesh = plsc.VectorSubcoreMesh(core_axis_name="c", subcore_axis_name="s")
    CSL = C * SL

    @functools.partial(
        pl.kernel,
        out_type=jax.ShapeDtypeStruct((B, OUTW), jnp.float32),
        mesh=mesh,
        compiler_params=pltpu.CompilerParams(use_tc_tiling_on_sc=False),
        scratch_types=[
            pltpu.VMEM((CSL + 16,), jnp.int32),
            pltpu.VMEM((CSL + 16,), jnp.int32),
        ] + [pltpu.VMEM((SL, D), jnp.float32) for _ in range(RD)]
          + [pltpu.VMEM((C, OUTW), jnp.float32),
             pltpu.SemaphoreType.DMA]
          + [pltpu.SemaphoreType.DMA for _ in range(RD)],
    )
    def k(ids_hbm, emb_hbm, out_hbm, ids_a, ids_b, *rest):
        rows = rest[:RD]
        out_b = rest[RD]
        sem_i = rest[RD + 1]
        gsems = rest[RD + 2:]
        wid = lax.axis_index("c") * NS + lax.axis_index("s")
        wbase = wid * SEQ_PER_W
        lane = lax.iota(jnp.int32, 16)
        zeros = jnp.zeros((16,), jnp.float32)
        tail_lanes = SL - (SL // 16) * 16

        def fire_ids(i, buf):
            pltpu.async_copy(
                ids_hbm.at[pl.ds((wbase + i * C) * SL, CSL)],
                buf.at[pl.ds(0, CSL)], sem_i)

        def wait_ids(buf):
            pltpu.make_async_copy(
                ids_hbm.at[pl.ds(0, CSL)], buf.at[pl.ds(0, CSL)], sem_i).wait()

        def fire_gather(ids_buf, s, rbuf, sem):
            pltpu.async_copy(
                emb_hbm.at[ids_buf.at[pl.ds(s * SL, SL)]], rbuf, sem)

        def wait_gather(rbuf, sem):
            pltpu.make_async_copy(emb_hbm.at[pl.ds(0, SL)], rbuf, sem).wait()

        def process_seq(ids_buf, s, rbuf):
            def red_body(j, carry):
                a0, a1 = carry
                r = j * 8
                for u in range(8):
                    a0 = a0 + rbuf[r + u, pl.ds(0, 16)]
                    a1 = a1 + rbuf[r + u, pl.ds(16, 16)]
                return a0, a1

            a0, a1 = lax.fori_loop(0, SL // 8, red_body, (zeros, zeros))

            def cnt_body(kk, cv):
                ids16 = ids_buf[pl.ds(s * SL + kk * 16, 16)]
                return cv + jnp.where(ids16 != 0, 1.0, 0.0).astype(jnp.float32)

            cv = lax.fori_loop(0, SL // 16, cnt_body, zeros)
            ids_tail = ids_buf[pl.ds(s * SL + (SL // 16) * 16, 16)]
            tail_ok = (lane < tail_lanes) & (ids_tail != 0)
            cv = cv + jnp.where(tail_ok, 1.0, 0.0).astype(jnp.float32)

            out_b[s, pl.ds(0, 16)] = a0
            out_b[s, pl.ds(16, 16)] = a1
            out_b[s, pl.ds(32, 16)] = cv

        def chunk_body(i, ids_buf, ids_other):
            wait_ids(ids_buf)

            @pl.when(i + 1 < NCHUNK)
            def _():
                fire_ids(i + 1, ids_other)

            for p in range(RD - 1):
                fire_gather(ids_buf, p, rows[p], gsems[p])

            def pipe_body(h, _):
                s0 = RD * h
                fire_gather(ids_buf, s0 + RD - 1, rows[RD - 1], gsems[RD - 1])
                for p in range(RD):
                    rbuf, sem = rows[p], gsems[p]
                    wait_gather(rbuf, sem)
                    process_seq(ids_buf, s0 + p, rbuf)
                    if p < RD - 1:
                        @pl.when(s0 + RD + p < C)
                        def _(rbuf=rbuf, sem=sem, p=p):
                            fire_gather(ids_buf, s0 + RD + p, rbuf, sem)
                return 0

            lax.fori_loop(0, C // RD, pipe_body, 0)
            pltpu.sync_copy(out_b, out_hbm.at[pl.ds(wbase + i * C, C)])

        fire_ids(0, ids_a)

        def outer_body(h, _):
            chunk_body(2 * h, ids_a, ids_b)
            chunk_body(2 * h + 1, ids_b, ids_a)
            return 0

        lax.fori_loop(0, NCHUNK // 2, outer_body, 0)

    return k(ids4, emb_lin)


def _tc_project(z, W, b2d):
    BLK = 4096

    def body(z_ref, w_ref, b_ref, o_ref):
        zz = z_ref[...]
        s = zz[:, :D]
        cnt = jnp.sum(zz[:, D:D + 16], axis=1, keepdims=True)
        acc = jnp.dot(s, w_ref[...], preferred_element_type=jnp.float32)
        o_ref[...] = acc / (cnt + 1e-10) + b_ref[...]

    return pl.pallas_call(
        body,
        grid=(B // BLK,),
        in_specs=[
            pl.BlockSpec((BLK, OUTW), lambda i: (i, 0)),
            pl.BlockSpec((D, PROJ), lambda i: (0, 0)),
            pl.BlockSpec((1, PROJ), lambda i: (0, 0)),
        ],
        out_specs=pl.BlockSpec((BLK, PROJ), lambda i: (i, 0)),
        out_shape=jax.ShapeDtypeStruct((B, PROJ), jnp.float32),
    )(z, W, b2d)


def kernel(input_ids, emb, W, b):
    emb_pad = _tc_relayout(emb.T)
    emb_lin = emb_pad.reshape(-1, D)
    ii = input_ids.reshape(-1).astype(jnp.int32)
    ids4 = (((ii >> 16) << 16) + ((ii & (OBK - 1)) << 2)
            + ((ii >> 14) & 3))
    z = _sc_pool(ids4, emb_lin)
    return _tc_project(z, W, b.reshape(1, PROJ))
